# baseline (speedup 1.0000x reference)
import jax
import jax.numpy as jnp
from jax.experimental import pallas as pl

B, N_PER, E_PER, D, R_PER, C_DIM = 8, 1250, 40000, 128, 625, 4


def _merge_body(x_ref, sh_ref, sp_ref, cp_ref, e_ref,
                ox_ref, oe_ref, osh_ref, osp_ref, ocp_ref):
    b = pl.program_id(0)
    ox_ref[...] = x_ref[...]
    oe_ref[:, 0, 0, :] = e_ref[0] + b * N_PER
    osh_ref[...] = sh_ref[...]
    osp_ref[...] = sp_ref[...]
    ocp_ref[...] = cp_ref[...]


def kernel(x, shift, shape, coupling, edge_index):
    shift3 = shift.reshape(B, 1, R_PER)
    shape3 = shape.reshape(B, 1, R_PER)
    coupling3 = coupling.reshape(B, 1, R_PER * C_DIM)

    out_x, out_e, out_sh, out_sp, out_cp = pl.pallas_call(
        _merge_body,
        grid=(B,),
        in_specs=[
            pl.BlockSpec((1, N_PER, D), lambda b: (b, 0, 0)),
            pl.BlockSpec((1, 1, R_PER), lambda b: (b, 0, 0)),
            pl.BlockSpec((1, 1, R_PER), lambda b: (b, 0, 0)),
            pl.BlockSpec((1, 1, R_PER * C_DIM), lambda b: (b, 0, 0)),
            pl.BlockSpec((1, 2, E_PER), lambda b: (b, 0, 0)),
        ],
        out_specs=[
            pl.BlockSpec((1, N_PER, D), lambda b: (b, 0, 0)),
            pl.BlockSpec((2, 1, 1, E_PER), lambda b: (0, b, 0, 0)),
            pl.BlockSpec((1, 1, R_PER), lambda b: (b, 0, 0)),
            pl.BlockSpec((1, 1, R_PER), lambda b: (b, 0, 0)),
            pl.BlockSpec((1, 1, R_PER * C_DIM), lambda b: (b, 0, 0)),
        ],
        out_shape=[
            jax.ShapeDtypeStruct((B, N_PER, D), jnp.float32),
            jax.ShapeDtypeStruct((2, B, 1, E_PER), jnp.int32),
            jax.ShapeDtypeStruct((B, 1, R_PER), jnp.float32),
            jax.ShapeDtypeStruct((B, 1, R_PER), jnp.float32),
            jax.ShapeDtypeStruct((B, 1, R_PER * C_DIM), jnp.float32),
        ],
    )(x, shift3, shape3, coupling3, edge_index)

    return (
        out_x.reshape(B * N_PER, D),
        out_e.reshape(2, B * E_PER),
        out_sh.reshape(B * R_PER),
        out_sp.reshape(B * R_PER),
        out_cp.reshape(B * R_PER, C_DIM),
    )


# speedup vs baseline: 1.0575x; 1.0575x over previous
import jax
import jax.numpy as jnp
from jax.experimental import pallas as pl

B, N_PER, E_PER, D, R_PER, C_DIM = 8, 1250, 40000, 128, 625, 4
E_SUB, E_LANE = 40, 1000


def _merge_body(x_ref, sh_ref, sp_ref, cp_ref, e_ref,
                ox_ref, oe_ref, osh_ref, osp_ref, ocp_ref):
    b = pl.program_id(0)
    ox_ref[...] = x_ref[...]
    oe_ref[:, 0] = e_ref[0] + b * N_PER
    osh_ref[...] = sh_ref[...]
    osp_ref[...] = sp_ref[...]
    ocp_ref[...] = cp_ref[...]


def kernel(x, shift, shape, coupling, edge_index):
    shift3 = shift.reshape(B, 1, R_PER)
    shape3 = shape.reshape(B, 1, R_PER)
    coupling3 = coupling.reshape(B, 1, R_PER * C_DIM)
    edges4 = edge_index.reshape(B, 2, E_SUB, E_LANE)

    out_x, out_e, out_sh, out_sp, out_cp = pl.pallas_call(
        _merge_body,
        grid=(B,),
        in_specs=[
            pl.BlockSpec((1, N_PER, D), lambda b: (b, 0, 0)),
            pl.BlockSpec((1, 1, R_PER), lambda b: (b, 0, 0)),
            pl.BlockSpec((1, 1, R_PER), lambda b: (b, 0, 0)),
            pl.BlockSpec((1, 1, R_PER * C_DIM), lambda b: (b, 0, 0)),
            pl.BlockSpec((1, 2, E_SUB, E_LANE), lambda b: (b, 0, 0, 0)),
        ],
        out_specs=[
            pl.BlockSpec((1, N_PER, D), lambda b: (b, 0, 0)),
            pl.BlockSpec((2, 1, E_SUB, E_LANE), lambda b: (0, b, 0, 0)),
            pl.BlockSpec((1, 1, R_PER), lambda b: (b, 0, 0)),
            pl.BlockSpec((1, 1, R_PER), lambda b: (b, 0, 0)),
            pl.BlockSpec((1, 1, R_PER * C_DIM), lambda b: (b, 0, 0)),
        ],
        out_shape=[
            jax.ShapeDtypeStruct((B, N_PER, D), jnp.float32),
            jax.ShapeDtypeStruct((2, B, E_SUB, E_LANE), jnp.int32),
            jax.ShapeDtypeStruct((B, 1, R_PER), jnp.float32),
            jax.ShapeDtypeStruct((B, 1, R_PER), jnp.float32),
            jax.ShapeDtypeStruct((B, 1, R_PER * C_DIM), jnp.float32),
        ],
    )(x, shift3, shape3, coupling3, edges4)

    return (
        out_x.reshape(B * N_PER, D),
        out_e.reshape(2, B * E_PER),
        out_sh.reshape(B * R_PER),
        out_sp.reshape(B * R_PER),
        out_cp.reshape(B * R_PER, C_DIM),
    )
